# Initial kernel scaffold; baseline (speedup 1.0000x reference)
import jax
import jax.numpy as jnp
from jax.experimental import pallas as pl


def kernel(head_embeddings, tail_embeddings, relation_embeddings, positive_samples, negative_samples):
    raise NotImplementedError("write your pallas kernel here")


# speedup vs baseline: 1.0754x; 1.0754x over previous
import functools

import jax
import jax.numpy as jnp
from jax import lax
from jax.experimental import pallas as pl
from jax.experimental.pallas import tpu as pltpu
from jax.experimental.pallas import tpu_sc as plsc

_MARGIN = 1.0
_LANES = 16


@functools.lru_cache(maxsize=None)
def _build_sc_loss(num_nodes, num_rel, dim, batch, num_neg):
    info = plsc.get_sparse_core_info()
    nc, ns = info.num_cores, info.num_subcores
    nw = nc * ns
    assert dim % _LANES == 0
    assert batch % (nw * _LANES) == 0
    per_w = batch // nw
    p = _LANES
    n_chunks = per_w // p
    pn = p * num_neg
    dchunks = dim // _LANES

    mesh = plsc.VectorSubcoreMesh(core_axis_name="c", subcore_axis_name="s")

    @functools.partial(
        pl.kernel,
        out_type=jax.ShapeDtypeStruct((nw * _LANES,), jnp.float32),
        mesh=mesh,
        compiler_params=pltpu.CompilerParams(needs_layout_passes=False),
        scratch_types=[
            pltpu.VMEM((p,), jnp.int32),
            pltpu.VMEM((p,), jnp.int32),
            pltpu.VMEM((p,), jnp.int32),
            pltpu.VMEM((pn,), jnp.int32),
            pltpu.VMEM((pn,), jnp.int32),
            pltpu.VMEM((pn,), jnp.int32),
            pltpu.VMEM((p, dim), jnp.float32),
            pltpu.VMEM((p, dim), jnp.float32),
            pltpu.VMEM((p, dim), jnp.float32),
            pltpu.VMEM((pn, dim), jnp.float32),
            pltpu.VMEM((pn, dim), jnp.float32),
            pltpu.VMEM((pn, dim), jnp.float32),
            pltpu.VMEM((_LANES,), jnp.float32),
            pltpu.SemaphoreType.DMA,
        ],
    )
    def sc_loss(h_hbm, t_hbm, r_hbm,
                ph_hbm, pr_hbm, pt_hbm, nh_hbm, nr_hbm, nt_hbm,
                out_hbm,
                iph, ipr, ipt, inh, inr, int_,
                ph, prr, pt, nh, nr, nt, ob, sem):
        wid = lax.axis_index("s") * nc + lax.axis_index("c")
        pos_base = wid * per_w
        neg_base = wid * per_w * num_neg
        lane = lax.iota(jnp.int32, _LANES)

        def triple_score(hbuf, rbuf, tbuf, row):
            acc = (hbuf[row, pl.ds(0, _LANES)]
                   * rbuf[row, pl.ds(0, _LANES)]
                   * tbuf[row, pl.ds(0, _LANES)])
            for cc in range(1, dchunks):
                acc = acc + (hbuf[row, pl.ds(cc * _LANES, _LANES)]
                             * rbuf[row, pl.ds(cc * _LANES, _LANES)]
                             * tbuf[row, pl.ds(cc * _LANES, _LANES)])
            return jnp.sum(acc)

        def chunk(c, loss_acc):
            pb = pos_base + c * p
            nb = neg_base + c * pn
            pltpu.sync_copy(ph_hbm.at[pl.ds(pb, p)], iph)
            pltpu.sync_copy(pr_hbm.at[pl.ds(pb, p)], ipr)
            pltpu.sync_copy(pt_hbm.at[pl.ds(pb, p)], ipt)
            pltpu.sync_copy(nh_hbm.at[pl.ds(nb, pn)], inh)
            pltpu.sync_copy(nr_hbm.at[pl.ds(nb, pn)], inr)
            pltpu.sync_copy(nt_hbm.at[pl.ds(nb, pn)], int_)
            cps = [
                pltpu.async_copy(h_hbm.at[iph], ph, sem),
                pltpu.async_copy(r_hbm.at[ipr], prr, sem),
                pltpu.async_copy(t_hbm.at[ipt], pt, sem),
                pltpu.async_copy(h_hbm.at[inh], nh, sem),
                pltpu.async_copy(r_hbm.at[inr], nr, sem),
                pltpu.async_copy(t_hbm.at[int_], nt, sem),
            ]
            for cp in cps:
                cp.wait()

            def pscore(i, acc_vec):
                s = triple_score(ph, prr, pt, i)
                return jnp.where(lane == i, s, acc_vec)

            pos_s = lax.fori_loop(0, p, pscore, jnp.zeros((_LANES,), jnp.float32))

            la = loss_acc
            for n in range(num_neg):
                def nscore(i, acc_vec, _n=n):
                    s = triple_score(nh, nr, nt, i * num_neg + _n)
                    return jnp.where(lane == i, s, acc_vec)

                neg_s = lax.fori_loop(0, p, nscore,
                                      jnp.zeros((_LANES,), jnp.float32))
                la = la + jnp.maximum(_MARGIN - pos_s + neg_s, 0.0)
            return la

        loss = lax.fori_loop(0, n_chunks, chunk,
                             jnp.zeros((_LANES,), jnp.float32))
        ob[...] = loss
        pltpu.sync_copy(ob, out_hbm.at[pl.ds(wid * _LANES, _LANES)])

    return sc_loss


def kernel(head_embeddings, tail_embeddings, relation_embeddings,
           positive_samples, negative_samples):
    batch, num_neg = negative_samples.shape[:2]
    num_nodes, dim = head_embeddings.shape
    num_rel = relation_embeddings.shape[0]

    ph = positive_samples[:, 0]
    pr = positive_samples[:, 1]
    pt = positive_samples[:, 2]
    neg_flat = negative_samples.reshape(-1, 3)
    nh = neg_flat[:, 0]
    nr = neg_flat[:, 1]
    nt = neg_flat[:, 2]

    fn = _build_sc_loss(num_nodes, num_rel, dim, batch, num_neg)
    partials = fn(head_embeddings, tail_embeddings, relation_embeddings,
                  ph, pr, pt, nh, nr, nt)
    return jnp.sum(partials) / (batch * num_neg)


# speedup vs baseline: 2.3613x; 2.1958x over previous
import functools

import jax
import jax.numpy as jnp
from jax import lax
from jax.experimental import pallas as pl
from jax.experimental.pallas import tpu as pltpu
from jax.experimental.pallas import tpu_sc as plsc

_MARGIN = 1.0
_LANES = 16


@functools.lru_cache(maxsize=None)
def _build_sc_loss(num_nodes, num_rel, dim, batch, num_neg):
    info = plsc.get_sparse_core_info()
    nc, ns = info.num_cores, info.num_subcores
    nw = nc * ns
    assert dim % _LANES == 0
    assert batch % (nw * _LANES) == 0
    per_w = batch // nw
    p = _LANES
    n_chunks = per_w // p
    rows = p * (1 + num_neg)
    assert rows <= 128
    idx_per_w = n_chunks * rows
    dchunks = dim // _LANES

    mesh = plsc.VectorSubcoreMesh(core_axis_name="c", subcore_axis_name="s")

    @functools.partial(
        pl.kernel,
        out_type=jax.ShapeDtypeStruct((nw * _LANES,), jnp.float32),
        mesh=mesh,
        compiler_params=pltpu.CompilerParams(needs_layout_passes=False),
        scratch_types=[
            pltpu.VMEM((idx_per_w,), jnp.int32),
            pltpu.VMEM((idx_per_w,), jnp.int32),
            pltpu.VMEM((idx_per_w,), jnp.int32),
            pltpu.VMEM((2, rows, dim), jnp.float32),
            pltpu.VMEM((2, rows, dim), jnp.float32),
            pltpu.VMEM((2, rows, dim), jnp.float32),
            pltpu.VMEM((_LANES,), jnp.float32),
            pltpu.SemaphoreType.DMA,
            pltpu.SemaphoreType.DMA,
        ],
    )
    def sc_loss(h_hbm, t_hbm, r_hbm, hi_hbm, ri_hbm, ti_hbm, out_hbm,
                ih, ir, it, hb, rb, tb, ob, sem0, sem1):
        wid = lax.axis_index("s") * nc + lax.axis_index("c")
        idx_base = wid * idx_per_w
        lane = lax.iota(jnp.int32, _LANES)
        sems = (sem0, sem1)

        pltpu.sync_copy(hi_hbm.at[pl.ds(idx_base, idx_per_w)], ih)
        pltpu.sync_copy(ri_hbm.at[pl.ds(idx_base, idx_per_w)], ir)
        pltpu.sync_copy(ti_hbm.at[pl.ds(idx_base, idx_per_w)], it)

        def copies(c, b):
            off = c * rows
            return (
                pltpu.make_async_copy(
                    h_hbm.at[ih.at[pl.ds(off, rows)]], hb.at[b], sems[b]),
                pltpu.make_async_copy(
                    r_hbm.at[ir.at[pl.ds(off, rows)]], rb.at[b], sems[b]),
                pltpu.make_async_copy(
                    t_hbm.at[it.at[pl.ds(off, rows)]], tb.at[b], sems[b]),
            )

        def fire(c, b):
            for cp in copies(c, b):
                cp.start()

        def wait(c, b):
            for cp in copies(c, b):
                cp.wait()

        def triple_score(b, row):
            acc = (hb[b, row, pl.ds(0, _LANES)]
                   * rb[b, row, pl.ds(0, _LANES)]
                   * tb[b, row, pl.ds(0, _LANES)])
            for cc in range(1, dchunks):
                acc = acc + (hb[b, row, pl.ds(cc * _LANES, _LANES)]
                             * rb[b, row, pl.ds(cc * _LANES, _LANES)]
                             * tb[b, row, pl.ds(cc * _LANES, _LANES)])
            return jnp.sum(acc)

        def compute(b, loss_acc):
            def pscore(i, acc_vec):
                s = triple_score(b, i)
                return jnp.where(lane == i, s, acc_vec)

            pos_s = lax.fori_loop(0, p, pscore,
                                  jnp.zeros((_LANES,), jnp.float32))
            la = loss_acc
            for n in range(num_neg):
                def nscore(i, acc_vec, _n=n):
                    s = triple_score(b, p + i * num_neg + _n)
                    return jnp.where(lane == i, s, acc_vec)

                neg_s = lax.fori_loop(0, p, nscore,
                                      jnp.zeros((_LANES,), jnp.float32))
                la = la + jnp.maximum(_MARGIN - pos_s + neg_s, 0.0)
            return la

        fire(0, 0)

        def outer(c2, loss_acc):
            la = loss_acc
            for b in range(2):
                c = c2 * 2 + b

                @pl.when(c + 1 < n_chunks)
                def _():
                    fire(c + 1, 1 - b)

                wait(c, b)
                la = compute(b, la)
            return la

        loss = lax.fori_loop(0, n_chunks // 2, outer,
                             jnp.zeros((_LANES,), jnp.float32))
        ob[...] = loss
        pltpu.sync_copy(ob, out_hbm.at[pl.ds(wid * _LANES, _LANES)])

    return sc_loss


def kernel(head_embeddings, tail_embeddings, relation_embeddings,
           positive_samples, negative_samples):
    batch, num_neg = negative_samples.shape[:2]
    num_nodes, dim = head_embeddings.shape
    num_rel = relation_embeddings.shape[0]
    p = _LANES
    n_chunks_total = batch // p

    pos3 = positive_samples.reshape(n_chunks_total, p, 3)
    neg3 = negative_samples.reshape(n_chunks_total, p * num_neg, 3)
    comb = jnp.concatenate([pos3, neg3], axis=1)
    hidx = comb[:, :, 0].reshape(-1)
    ridx = comb[:, :, 1].reshape(-1)
    tidx = comb[:, :, 2].reshape(-1)

    fn = _build_sc_loss(num_nodes, num_rel, dim, batch, num_neg)
    partials = fn(head_embeddings, tail_embeddings, relation_embeddings,
                  hidx, ridx, tidx)
    return jnp.sum(partials) / (batch * num_neg)


# speedup vs baseline: 2.3861x; 1.0105x over previous
import functools

import jax
import jax.numpy as jnp
from jax import lax
from jax.experimental import pallas as pl
from jax.experimental.pallas import tpu as pltpu
from jax.experimental.pallas import tpu_sc as plsc

_MARGIN = 1.0
_LANES = 16


@functools.lru_cache(maxsize=None)
def _build_sc_loss(num_nodes, num_rel, dim, batch, num_neg):
    info = plsc.get_sparse_core_info()
    nc, ns = info.num_cores, info.num_subcores
    nw = nc * ns
    assert dim % _LANES == 0
    assert batch % (nw * _LANES) == 0
    per_w = batch // nw
    p = _LANES
    n_chunks = per_w // p
    rows = p * (1 + num_neg)
    assert rows <= 128
    idx_per_w = n_chunks * rows
    dchunks = dim // _LANES

    mesh = plsc.VectorSubcoreMesh(core_axis_name="c", subcore_axis_name="s")

    @functools.partial(
        pl.kernel,
        out_type=jax.ShapeDtypeStruct((nw * _LANES,), jnp.float32),
        mesh=mesh,
        compiler_params=pltpu.CompilerParams(needs_layout_passes=False),
        scratch_types=[
            pltpu.VMEM((idx_per_w,), jnp.int32),
            pltpu.VMEM((idx_per_w,), jnp.int32),
            pltpu.VMEM((idx_per_w,), jnp.int32),
            pltpu.VMEM((2, rows, dim), jnp.float32),
            pltpu.VMEM((2, rows, dim), jnp.float32),
            pltpu.VMEM((2, rows, dim), jnp.float32),
            pltpu.VMEM((_LANES,), jnp.float32),
            pltpu.SemaphoreType.DMA,
            pltpu.SemaphoreType.DMA,
        ],
    )
    def sc_loss(h_hbm, t_hbm, r_hbm, hi_hbm, ri_hbm, ti_hbm, out_hbm,
                ih, ir, it, hb, rb, tb, ob, sem0, sem1):
        wid = lax.axis_index("s") * nc + lax.axis_index("c")
        idx_base = wid * idx_per_w
        lane = lax.iota(jnp.int32, _LANES)
        sems = (sem0, sem1)

        preloads = [
            pltpu.make_async_copy(
                src.at[pl.ds(idx_base, idx_per_w)], dst, sem0)
            for src, dst in ((hi_hbm, ih), (ri_hbm, ir), (ti_hbm, it))
        ]
        for cp in preloads:
            cp.start()
        for cp in preloads:
            cp.wait()

        def copies(c, b):
            off = c * rows
            return (
                pltpu.make_async_copy(
                    h_hbm.at[ih.at[pl.ds(off, rows)]], hb.at[b], sems[b]),
                pltpu.make_async_copy(
                    r_hbm.at[ir.at[pl.ds(off, rows)]], rb.at[b], sems[b]),
                pltpu.make_async_copy(
                    t_hbm.at[it.at[pl.ds(off, rows)]], tb.at[b], sems[b]),
            )

        def fire(c, b):
            for cp in copies(c, b):
                cp.start()

        def wait(c, b):
            for cp in copies(c, b):
                cp.wait()

        def triple_score(b, row):
            acc = (hb[b, row, pl.ds(0, _LANES)]
                   * rb[b, row, pl.ds(0, _LANES)]
                   * tb[b, row, pl.ds(0, _LANES)])
            for cc in range(1, dchunks):
                acc = acc + (hb[b, row, pl.ds(cc * _LANES, _LANES)]
                             * rb[b, row, pl.ds(cc * _LANES, _LANES)]
                             * tb[b, row, pl.ds(cc * _LANES, _LANES)])
            return jnp.sum(acc)

        def compute(b, loss_acc):
            def pscore(i, acc_vec):
                s = triple_score(b, i)
                return jnp.where(lane == i, s, acc_vec)

            pos_s = lax.fori_loop(0, p, pscore,
                                  jnp.zeros((_LANES,), jnp.float32))
            la = loss_acc
            for n in range(num_neg):
                def nscore(i, acc_vec, _n=n):
                    s = triple_score(b, p + i * num_neg + _n)
                    return jnp.where(lane == i, s, acc_vec)

                neg_s = lax.fori_loop(0, p, nscore,
                                      jnp.zeros((_LANES,), jnp.float32))
                la = la + jnp.maximum(_MARGIN - pos_s + neg_s, 0.0)
            return la

        fire(0, 0)

        def outer(c2, loss_acc):
            la = loss_acc
            for b in range(2):
                c = c2 * 2 + b

                @pl.when(c + 1 < n_chunks)
                def _():
                    fire(c + 1, 1 - b)

                wait(c, b)
                la = compute(b, la)
            return la

        loss = lax.fori_loop(0, n_chunks // 2, outer,
                             jnp.zeros((_LANES,), jnp.float32))
        ob[...] = loss
        pltpu.sync_copy(ob, out_hbm.at[pl.ds(wid * _LANES, _LANES)])

    return sc_loss


def kernel(head_embeddings, tail_embeddings, relation_embeddings,
           positive_samples, negative_samples):
    batch, num_neg = negative_samples.shape[:2]
    num_nodes, dim = head_embeddings.shape
    num_rel = relation_embeddings.shape[0]
    p = _LANES
    n_chunks_total = batch // p

    pos3 = positive_samples.reshape(n_chunks_total, p, 3)
    neg3 = negative_samples.reshape(n_chunks_total, p * num_neg, 3)
    comb = jnp.concatenate([pos3, neg3], axis=1)
    hidx = comb[:, :, 0].reshape(-1)
    ridx = comb[:, :, 1].reshape(-1)
    tidx = comb[:, :, 2].reshape(-1)

    fn = _build_sc_loss(num_nodes, num_rel, dim, batch, num_neg)
    partials = fn(head_embeddings, tail_embeddings, relation_embeddings,
                  hidx, ridx, tidx)
    return jnp.sum(partials) / (batch * num_neg)
